# baseline (speedup 1.0000x reference)
import functools

import jax
import jax.numpy as jnp
from jax import lax
from jax.experimental import pallas as pl
from jax.experimental.pallas import tpu as pltpu
from jax.experimental.pallas import tpu_sc as plsc

B = 16384
L = 200
V = 100000
D = 128
H = 64

_NC = 2
_NS = 16
_NW = _NC * _NS
_BPW = B // _NW
_C0 = 104
_C1 = 96
_BB = 16
_NBLK = _BPW // _BB
_NBUF = 8


def _t2_body(t_ref, w_ref, o_ref):
    o_ref[...] = (jnp.dot(t_ref[...], w_ref[...],
                          preferred_element_type=jnp.float32)
                  * (1.0 / L)).astype(jnp.bfloat16)


_t2_call = pl.pallas_call(
    _t2_body,
    grid=(50,),
    in_specs=[pl.BlockSpec((V // 50, D), lambda i: (i, 0)),
              pl.BlockSpec((D, H), lambda i: (0, 0))],
    out_specs=pl.BlockSpec((V // 50, H), lambda i: (i, 0)),
    out_shape=jax.ShapeDtypeStruct((V, H), jnp.bfloat16),
)


def _make_sc_pool():
    mesh = plsc.VectorSubcoreMesh(core_axis_name="c", subcore_axis_name="s")

    @functools.partial(
        pl.kernel,
        mesh=mesh,
        compiler_params=pltpu.CompilerParams(use_tc_tiling_on_sc=False),
        out_type=jax.ShapeDtypeStruct((B, H), jnp.bfloat16),
        scratch_types=[
            pltpu.VMEM((2, _BB, L), jnp.int32),
            [pltpu.VMEM((L, H), jnp.bfloat16) for _ in range(_NBUF)],
            pltpu.VMEM((2, _BB, H), jnp.bfloat16),
            [pltpu.SemaphoreType.DMA for _ in range(_NBUF)],
            pltpu.SemaphoreType.DMA,
        ],
    )
    def sc_pool(ids_hbm, t2_hbm, out_hbm, ids_v, bufs, ob_v, gsems, i_sem):
        wid = lax.axis_index("s") * _NC + lax.axis_index("c")
        base = wid * _BPW

        def issue(r, buf, sem):
            slot = (r // _BB) % 2
            rr = r % _BB
            pltpu.async_copy(
                t2_hbm.at[ids_v.at[slot, rr, pl.ds(0, _C0)]],
                buf.at[pl.ds(0, _C0)], sem)
            pltpu.async_copy(
                t2_hbm.at[ids_v.at[slot, rr, pl.ds(_C0, _C1)]],
                buf.at[pl.ds(_C0, _C1)], sem)

        def drain(buf, sem):
            pltpu.make_async_copy(
                t2_hbm.at[ids_v.at[0, 0, pl.ds(0, _C0)]],
                buf.at[pl.ds(0, _C0)], sem).wait()
            pltpu.make_async_copy(
                t2_hbm.at[ids_v.at[0, 0, pl.ds(0, _C1)]],
                buf.at[pl.ds(_C0, _C1)], sem).wait()

        def accum(buf, r):
            def body(k, accs):
                a0, a1 = accs
                lb = k * 8
                for dl in range(8):
                    l = lb + dl
                    a0 = a0 + buf[l, pl.ds(0, 32)]
                    a1 = a1 + buf[l, pl.ds(32, 32)]
                return a0, a1

            z = jnp.zeros((32,), jnp.bfloat16)
            a0, a1 = lax.fori_loop(0, L // 8, body, (z, z))
            pblk = (r // _BB) % 2
            rr = r % _BB
            ob_v[pblk, rr, pl.ds(0, 32)] = a0
            ob_v[pblk, rr, pl.ds(32, 32)] = a1

        pltpu.sync_copy(ids_hbm.at[pl.ds(base, _BB)], ids_v.at[0])
        pltpu.async_copy(ids_hbm.at[pl.ds(base + _BB, _BB)], ids_v.at[1], i_sem)
        for k in range(_NBUF):
            issue(k, bufs[k], gsems[k])

        def per_group(i, carry):
            r0 = _NBUF * i
            nxt = r0 + _NBUF

            for k in range(_NBUF):
                drain(bufs[k], gsems[k])
                accum(bufs[k], r0 + k)

            @pl.when(jnp.logical_and(nxt % _BB == 0, nxt < _BPW))
            def _():
                pltpu.make_async_copy(
                    ids_hbm.at[pl.ds(base, _BB)], ids_v.at[0], i_sem).wait()

            @pl.when(nxt < _BPW)
            def _():
                for k in range(_NBUF):
                    issue(nxt + k, bufs[k], gsems[k])

            @pl.when(jnp.logical_and(nxt % _BB == 0, nxt + _BB < _BPW))
            def _():
                nblk = nxt // _BB
                pltpu.async_copy(
                    ids_hbm.at[pl.ds(base + (nblk + 1) * _BB, _BB)],
                    ids_v.at[(nblk + 1) % 2], i_sem)

            @pl.when(nxt % _BB == 0)
            def _():
                blk = r0 // _BB
                pltpu.sync_copy(
                    ob_v.at[blk % 2],
                    out_hbm.at[pl.ds(base + blk * _BB, _BB)])

            return carry

        lax.fori_loop(0, _BPW // _NBUF, per_group, 0)

    return sc_pool


_sc_pool = _make_sc_pool()


def _head_body(h_ref, b1_ref, w2_ref, b2_ref, o_ref):
    h = jnp.maximum(h_ref[...].astype(jnp.float32) + b1_ref[...], 0.0)
    logits = jnp.dot(h, w2_ref[...], preferred_element_type=jnp.float32)
    o_ref[...] = jax.nn.sigmoid(logits + b2_ref[...])[:, 0]


_head_call = pl.pallas_call(
    _head_body,
    grid=(8,),
    in_specs=[pl.BlockSpec((B // 8, H), lambda i: (i, 0)),
              pl.BlockSpec((1, H), lambda i: (0, 0)),
              pl.BlockSpec((H, 1), lambda i: (0, 0)),
              pl.BlockSpec((1, 1), lambda i: (0, 0))],
    out_specs=pl.BlockSpec((B // 8,), lambda i: (i,)),
    out_shape=jax.ShapeDtypeStruct((B,), jnp.float32),
)


def kernel(input_ids, table, W1, b1, W2, b2):
    ids = input_ids.astype(jnp.int32)
    t2 = _t2_call(table, W1)
    hsum = _sc_pool(ids, t2)
    return _head_call(hsum, b1.reshape(1, H), W2, b2.reshape(1, 1))


# speedup vs baseline: 1.3370x; 1.3370x over previous
import functools

import jax
import jax.numpy as jnp
from jax import lax
from jax.experimental import pallas as pl
from jax.experimental.pallas import tpu as pltpu
from jax.experimental.pallas import tpu_sc as plsc

B = 16384
L = 200
V = 100000
D = 128
H = 64

_NC = 2
_NS = 16
_NW = _NC * _NS
_BPW = B // _NW
_C0 = 104
_C1 = 96
_BB = 16
_NBLK = _BPW // _BB
_NBUF = 8


def _t2_body(t_ref, w_ref, o_ref):
    o_ref[...] = (jnp.dot(t_ref[...], w_ref[...],
                          preferred_element_type=jnp.float32)
                  * (1.0 / L)).astype(jnp.bfloat16)


_t2_call = pl.pallas_call(
    _t2_body,
    grid=(50,),
    in_specs=[pl.BlockSpec((V // 50, D), lambda i: (i, 0)),
              pl.BlockSpec((D, H), lambda i: (0, 0))],
    out_specs=pl.BlockSpec((V // 50, H), lambda i: (i, 0)),
    out_shape=jax.ShapeDtypeStruct((V, H), jnp.bfloat16),
)


def _make_sc_pool():
    mesh = plsc.VectorSubcoreMesh(core_axis_name="c", subcore_axis_name="s")

    @functools.partial(
        pl.kernel,
        mesh=mesh,
        compiler_params=pltpu.CompilerParams(use_tc_tiling_on_sc=False),
        out_type=jax.ShapeDtypeStruct((B, H), jnp.bfloat16),
        scratch_types=[
            pltpu.VMEM((2, _BB, L), jnp.int32),
            [pltpu.VMEM((L, H), jnp.bfloat16) for _ in range(_NBUF)],
            pltpu.VMEM((2, _BB, H), jnp.bfloat16),
            [pltpu.SemaphoreType.DMA for _ in range(_NBUF)],
            pltpu.SemaphoreType.DMA,
        ],
    )
    def sc_pool(ids_hbm, t2_hbm, out_hbm, ids_v, bufs, ob_v, gsems, i_sem):
        wid = lax.axis_index("s") * _NC + lax.axis_index("c")
        base = wid * _BPW

        def issue(r, buf, sem):
            slot = (r // _BB) % 2
            rr = r % _BB
            pltpu.async_copy(
                t2_hbm.at[ids_v.at[slot, rr, pl.ds(0, _C0)]],
                buf.at[pl.ds(0, _C0)], sem)
            pltpu.async_copy(
                t2_hbm.at[ids_v.at[slot, rr, pl.ds(_C0, _C1)]],
                buf.at[pl.ds(_C0, _C1)], sem)

        def drain(buf, sem):
            pltpu.make_async_copy(
                t2_hbm.at[ids_v.at[0, 0, pl.ds(0, _C0)]],
                buf.at[pl.ds(0, _C0)], sem).wait()
            pltpu.make_async_copy(
                t2_hbm.at[ids_v.at[0, 0, pl.ds(0, _C1)]],
                buf.at[pl.ds(_C0, _C1)], sem).wait()

        def accum(buf, r):
            def body(k, accs):
                a0, a1 = accs
                lb = k * 8
                for dl in range(8):
                    l = lb + dl
                    a0 = a0 + buf[l, pl.ds(0, 32)]
                    a1 = a1 + buf[l, pl.ds(32, 32)]
                return a0, a1

            z = jnp.zeros((32,), jnp.bfloat16)
            a0, a1 = lax.fori_loop(0, L // 8, body, (z, z))
            pblk = (r // _BB) % 2
            rr = r % _BB
            ob_v[pblk, rr, pl.ds(0, 32)] = a0
            ob_v[pblk, rr, pl.ds(32, 32)] = a1

        pltpu.sync_copy(ids_hbm.at[pl.ds(base, _BB)], ids_v.at[0])
        pltpu.async_copy(ids_hbm.at[pl.ds(base + _BB, _BB)], ids_v.at[1], i_sem)
        for k in range(_NBUF):
            issue(k, bufs[k], gsems[k])

        def per_group(i, carry):
            r0 = _NBUF * i
            nxt = r0 + _NBUF

            @pl.when(jnp.logical_and(nxt % _BB == 0, nxt < _BPW))
            def _():
                pltpu.make_async_copy(
                    ids_hbm.at[pl.ds(base, _BB)], ids_v.at[0], i_sem).wait()

            for k in range(_NBUF):
                drain(bufs[k], gsems[k])
                accum(bufs[k], r0 + k)

                @pl.when(nxt < _BPW)
                def _(k=k):
                    issue(nxt + k, bufs[k], gsems[k])

            @pl.when(jnp.logical_and(nxt % _BB == 0, nxt + _BB < _BPW))
            def _():
                nblk = nxt // _BB
                pltpu.async_copy(
                    ids_hbm.at[pl.ds(base + (nblk + 1) * _BB, _BB)],
                    ids_v.at[(nblk + 1) % 2], i_sem)

            @pl.when(nxt % _BB == 0)
            def _():
                blk = r0 // _BB
                pltpu.sync_copy(
                    ob_v.at[blk % 2],
                    out_hbm.at[pl.ds(base + blk * _BB, _BB)])

            return carry

        lax.fori_loop(0, _BPW // _NBUF, per_group, 0)

    return sc_pool


_sc_pool = _make_sc_pool()


def _head_body(h_ref, b1_ref, w2_ref, b2_ref, o_ref):
    h = jnp.maximum(h_ref[...].astype(jnp.float32) + b1_ref[...], 0.0)
    logits = jnp.dot(h, w2_ref[...], preferred_element_type=jnp.float32)
    o_ref[...] = jax.nn.sigmoid(logits + b2_ref[...])[:, 0]


_head_call = pl.pallas_call(
    _head_body,
    grid=(8,),
    in_specs=[pl.BlockSpec((B // 8, H), lambda i: (i, 0)),
              pl.BlockSpec((1, H), lambda i: (0, 0)),
              pl.BlockSpec((H, 1), lambda i: (0, 0)),
              pl.BlockSpec((1, 1), lambda i: (0, 0))],
    out_specs=pl.BlockSpec((B // 8,), lambda i: (i,)),
    out_shape=jax.ShapeDtypeStruct((B,), jnp.float32),
)


def kernel(input_ids, table, W1, b1, W2, b2):
    ids = input_ids.astype(jnp.int32)
    t2 = _t2_call(table, W1)
    hsum = _sc_pool(ids, t2)
    return _head_call(hsum, b1.reshape(1, H), W2, b2.reshape(1, 1))
